# baseline (speedup 1.0000x reference)
import functools

import jax
import jax.numpy as jnp
from jax import lax
from jax.experimental import pallas as pl
from jax.experimental.pallas import tpu as pltpu
from jax.experimental.pallas import tpu_sc as plsc

_NC = 2
_NS = 16
_CHUNKS = 4
_BB = 8


def _make_sc_gather(Bc, L, D):
    NW = _NC * _NS
    nb = Bc // NW

    chunks = []
    off = 0
    while off < L:
        c = min(128, L - off)
        chunks.append((off, c))
        off += c

    mesh = plsc.VectorSubcoreMesh(
        core_axis_name="c", subcore_axis_name="s",
        num_cores=_NC, num_subcores=_NS)

    @functools.partial(
        pl.kernel,
        mesh=mesh,
        out_type=jax.ShapeDtypeStruct((Bc, L, D), jnp.float32),
        scratch_types=[
            pltpu.VMEM((nb, L), jnp.int32),
            pltpu.VMEM((L, D), jnp.float32),
            pltpu.VMEM((L, D), jnp.float32),
            pltpu.SemaphoreType.DMA,
            pltpu.SemaphoreType.DMA,
            pltpu.SemaphoreType.DMA,
            pltpu.SemaphoreType.DMA,
        ],
    )
    def _gather(x_hbm, e_hbm, out_hbm, idx_v, rows0, rows1,
                gsem0, gsem1, wsem0, wsem1):
        wid = lax.axis_index("s") * _NC + lax.axis_index("c")
        base = wid * nb
        pltpu.sync_copy(x_hbm.at[pl.ds(base, nb)], idx_v)

        rows = (rows0, rows1)
        gsem = (gsem0, gsem1)
        wsem = (wsem0, wsem1)

        def start_gather(i, p):
            for off, c in chunks:
                pltpu.async_copy(e_hbm.at[idx_v.at[i, pl.ds(off, c)]],
                                 rows[p].at[pl.ds(off, c)], gsem[p])

        def wait_gather(p):
            pltpu.make_async_copy(e_hbm.at[pl.ds(0, L)], rows[p],
                                  gsem[p]).wait()

        def start_write(i, p):
            pltpu.async_copy(rows[p], out_hbm.at[base + i], wsem[p])

        def wait_write(p):
            pltpu.make_async_copy(rows[p], out_hbm.at[0], wsem[p]).wait()

        start_gather(0, 0)

        def body(i2, carry):
            for b2 in range(2):
                i = i2 * 2 + b2
                p = b2
                q = 1 - b2

                @pl.when(i >= 1)
                def _():
                    wait_write(q)

                @pl.when(i + 1 < nb)
                def _():
                    start_gather(i + 1, q)

                wait_gather(p)
                start_write(i, p)
            return carry

        lax.fori_loop(0, nb // 2, body, 0)
        wait_write((nb - 1) % 2)

    return _gather


def _ln_body(eg_ref, p_ref, g_ref, b_ref, o_ref):
    x = eg_ref[...] + p_ref[...]
    u = jnp.mean(x, axis=-1, keepdims=True)
    d = x - u
    v = jnp.mean(d * d, axis=-1, keepdims=True)
    o_ref[...] = g_ref[...] * (d * lax.rsqrt(v + 1e-12)) + b_ref[...]


def kernel(X, E_table, P_table, gamma, beta):
    B, L = X.shape
    V, D = E_table.shape
    X = X.astype(jnp.int32)
    Bc = B // _CHUNKS

    sc_gather = _make_sc_gather(Bc, L, D)
    P_L = P_table[:L]
    g2 = gamma.reshape(1, D)
    b2 = beta.reshape(1, D)

    tc_ln = pl.pallas_call(
        _ln_body,
        grid=(Bc // _BB,),
        in_specs=[
            pl.BlockSpec((_BB, L, D), lambda i: (i, 0, 0)),
            pl.BlockSpec((L, D), lambda i: (0, 0)),
            pl.BlockSpec((1, D), lambda i: (0, 0)),
            pl.BlockSpec((1, D), lambda i: (0, 0)),
        ],
        out_specs=pl.BlockSpec((_BB, L, D), lambda i: (i, 0, 0)),
        out_shape=jax.ShapeDtypeStruct((Bc, L, D), jnp.float32),
    )

    outs = []
    for c in range(_CHUNKS):
        eg = sc_gather(X[c * Bc:(c + 1) * Bc], E_table)
        outs.append(tc_ln(eg, P_L, g2, b2))
    return jnp.concatenate(outs, axis=0)


# speedup vs baseline: 1.4200x; 1.4200x over previous
import functools

import jax
import jax.numpy as jnp
from jax import lax
from jax.experimental import pallas as pl
from jax.experimental.pallas import tpu as pltpu
from jax.experimental.pallas import tpu_sc as plsc

_LANES = 16
_NC = 2
_NS = 16
_UNROLL = 4


def _allsum16(v):
    dnums = lax.GatherDimensionNumbers(
        offset_dims=(), collapsed_slice_dims=(0,), start_index_map=(0,))
    for k in (8, 4, 2, 1):
        idx = lax.iota(jnp.int32, _LANES) ^ k
        v = v + lax.gather(v, idx[:, None], dnums, slice_sizes=(1,),
                           mode=lax.GatherScatterMode.PROMISE_IN_BOUNDS)
    return v


def _rsqrt16(x):
    i = lax.bitcast_convert_type(x, jnp.int32)
    i = jnp.int32(0x5F3759DF) - lax.shift_right_logical(i, 1)
    y = lax.bitcast_convert_type(i, jnp.float32)
    for _ in range(1):
        y = y * (1.5 - 0.5 * x * y * y)
    return y


def kernel(X, E_table, P_table, gamma, beta):
    B, L = X.shape
    V, D = E_table.shape
    nd = D // _LANES
    NW = _NC * _NS
    nb = B // NW
    X = X.astype(jnp.int32)

    chunks = []
    off = 0
    while off < L:
        c = min(128, L - off)
        chunks.append((off, c))
        off += c

    mesh = plsc.VectorSubcoreMesh(
        core_axis_name="c", subcore_axis_name="s",
        num_cores=_NC, num_subcores=_NS)

    @functools.partial(
        pl.kernel,
        mesh=mesh,
        out_type=jax.ShapeDtypeStruct((B, L, D), jnp.float32),
        scratch_types=[
            pltpu.VMEM((nb, L), jnp.int32),
            pltpu.VMEM((L, D), jnp.float32),
            pltpu.VMEM((L, D), jnp.float32),
            pltpu.VMEM((L, D), jnp.float32),
            pltpu.VMEM((D,), jnp.float32),
            pltpu.VMEM((D,), jnp.float32),
            pltpu.SemaphoreType.DMA,
            pltpu.SemaphoreType.DMA,
            pltpu.SemaphoreType.DMA,
            pltpu.SemaphoreType.DMA,
        ],
    )
    def _emb_ln(x_hbm, e_hbm, p_hbm, g_hbm, b_hbm, out_hbm,
                idx_v, rows0, rows1, p_v, g_v, b_v,
                gsem0, gsem1, wsem0, wsem1):
        wid = lax.axis_index("s") * _NC + lax.axis_index("c")
        base = wid * nb

        pltpu.sync_copy(x_hbm.at[pl.ds(base, nb)], idx_v)
        pltpu.sync_copy(p_hbm.at[pl.ds(0, L)], p_v)
        pltpu.sync_copy(g_hbm, g_v)
        pltpu.sync_copy(b_hbm, b_v)
        gv = [g_v[pl.ds(j * _LANES, _LANES)] for j in range(nd)]
        bv = [b_v[pl.ds(j * _LANES, _LANES)] for j in range(nd)]

        rows = (rows0, rows1)
        gsem = (gsem0, gsem1)
        wsem = (wsem0, wsem1)

        def start_gather(i, p):
            for off, c in chunks:
                pltpu.async_copy(e_hbm.at[idx_v.at[i, pl.ds(off, c)]],
                                 rows[p].at[pl.ds(off, c)], gsem[p])

        def wait_gather(p):
            pltpu.make_async_copy(e_hbm.at[pl.ds(0, L)], rows[p],
                                  gsem[p]).wait()

        def start_write(i, p):
            pltpu.async_copy(rows[p], out_hbm.at[base + i], wsem[p])

        def wait_write(p):
            pltpu.make_async_copy(rows[p], out_hbm.at[0], wsem[p]).wait()

        def compute(p):
            rbuf = rows[p]

            @plsc.parallel_loop(0, L, unroll=_UNROLL)
            def row_body(r):
                xs = []
                s = None
                ss = None
                for j in range(nd):
                    e = rbuf[r, pl.ds(j * _LANES, _LANES)]
                    pj = p_v[r, pl.ds(j * _LANES, _LANES)]
                    x = e + pj
                    xs.append(x)
                    s = x if s is None else s + x
                    ss = x * x if ss is None else ss + x * x
                mean = _allsum16(s) * (1.0 / D)
                var = _allsum16(ss) * (1.0 / D) - mean * mean
                rstd = _rsqrt16(var + 1e-12)
                for j in range(nd):
                    o = (xs[j] - mean) * rstd * gv[j] + bv[j]
                    rbuf[r, pl.ds(j * _LANES, _LANES)] = o

        start_gather(0, 0)

        def body(i2, carry):
            for b2 in range(2):
                i = i2 * 2 + b2
                p = b2
                q = 1 - b2

                @pl.when(i >= 1)
                def _():
                    wait_write(q)

                @pl.when(i + 1 < nb)
                def _():
                    start_gather(i + 1, q)

                wait_gather(p)
                compute(p)
                start_write(i, p)
            return carry

        lax.fori_loop(0, nb // 2, body, 0)
        wait_write((nb - 1) % 2)

    return _emb_ln(X, E_table, P_table, gamma, beta)


# speedup vs baseline: 1.5990x; 1.1260x over previous
import functools

import jax
import jax.numpy as jnp
from jax import lax
from jax.experimental import pallas as pl
from jax.experimental.pallas import tpu as pltpu
from jax.experimental.pallas import tpu_sc as plsc

_LANES = 16
_NC = 2
_NS = 16
_UNROLL = 2


def _allsum16(v):
    dnums = lax.GatherDimensionNumbers(
        offset_dims=(), collapsed_slice_dims=(0,), start_index_map=(0,))
    for k in (8, 4, 2, 1):
        idx = lax.iota(jnp.int32, _LANES) ^ k
        v = v + lax.gather(v, idx[:, None], dnums, slice_sizes=(1,),
                           mode=lax.GatherScatterMode.PROMISE_IN_BOUNDS)
    return v


def _rsqrt16(x):
    i = lax.bitcast_convert_type(x, jnp.int32)
    i = jnp.int32(0x5F3759DF) - lax.shift_right_logical(i, 1)
    y = lax.bitcast_convert_type(i, jnp.float32)
    for _ in range(1):
        y = y * (1.5 - 0.5 * x * y * y)
    return y


def kernel(X, E_table, P_table, gamma, beta):
    B, L = X.shape
    V, D = E_table.shape
    nd = D // _LANES
    NW = _NC * _NS
    nb = B // NW
    X = X.astype(jnp.int32)

    chunks = []
    off = 0
    while off < L:
        c = min(128, L - off)
        chunks.append((off, c))
        off += c

    mesh = plsc.VectorSubcoreMesh(
        core_axis_name="c", subcore_axis_name="s",
        num_cores=_NC, num_subcores=_NS)

    @functools.partial(
        pl.kernel,
        mesh=mesh,
        out_type=jax.ShapeDtypeStruct((B, L, D), jnp.float32),
        scratch_types=[
            pltpu.VMEM((nb, L), jnp.int32),
            pltpu.VMEM((L, D), jnp.float32),
            pltpu.VMEM((L, D), jnp.float32),
            pltpu.VMEM((L, D), jnp.float32),
            pltpu.VMEM((D,), jnp.float32),
            pltpu.VMEM((D,), jnp.float32),
            pltpu.SemaphoreType.DMA,
            pltpu.SemaphoreType.DMA,
            pltpu.SemaphoreType.DMA,
            pltpu.SemaphoreType.DMA,
        ],
    )
    def _emb_ln(x_hbm, e_hbm, p_hbm, g_hbm, b_hbm, out_hbm,
                idx_v, rows0, rows1, p_v, g_v, b_v,
                gsem0, gsem1, wsem0, wsem1):
        wid = lax.axis_index("s") * _NC + lax.axis_index("c")
        base = wid * nb

        pltpu.sync_copy(x_hbm.at[pl.ds(base, nb)], idx_v)
        pltpu.sync_copy(p_hbm.at[pl.ds(0, L)], p_v)
        pltpu.sync_copy(g_hbm, g_v)
        pltpu.sync_copy(b_hbm, b_v)
        gv = [g_v[pl.ds(j * _LANES, _LANES)] for j in range(nd)]
        bv = [b_v[pl.ds(j * _LANES, _LANES)] for j in range(nd)]

        rows = (rows0, rows1)
        gsem = (gsem0, gsem1)
        wsem = (wsem0, wsem1)

        def start_gather(i, p):
            for off, c in chunks:
                pltpu.async_copy(e_hbm.at[idx_v.at[i, pl.ds(off, c)]],
                                 rows[p].at[pl.ds(off, c)], gsem[p])

        def wait_gather(p):
            pltpu.make_async_copy(e_hbm.at[pl.ds(0, L)], rows[p],
                                  gsem[p]).wait()

        def start_write(i, p):
            pltpu.async_copy(rows[p], out_hbm.at[base + i], wsem[p])

        def wait_write(p):
            pltpu.make_async_copy(rows[p], out_hbm.at[0], wsem[p]).wait()

        def compute(p):
            rbuf = rows[p]

            @plsc.parallel_loop(0, L, unroll=_UNROLL)
            def row_body(r):
                xs = []
                s = None
                ss = None
                for j in range(nd):
                    e = rbuf[r, pl.ds(j * _LANES, _LANES)]
                    pj = p_v[r, pl.ds(j * _LANES, _LANES)]
                    x = e + pj
                    xs.append(x)
                    s = x if s is None else s + x
                    ss = x * x if ss is None else ss + x * x
                mean = _allsum16(s) * (1.0 / D)
                var = _allsum16(ss) * (1.0 / D) - mean * mean
                rstd = _rsqrt16(var + 1e-12)
                for j in range(nd):
                    o = (xs[j] - mean) * rstd * gv[j] + bv[j]
                    rbuf[r, pl.ds(j * _LANES, _LANES)] = o

        start_gather(0, 0)

        def body(i2, carry):
            for b2 in range(2):
                i = i2 * 2 + b2
                p = b2
                q = 1 - b2

                @pl.when(i >= 1)
                def _():
                    wait_write(q)

                @pl.when(i + 1 < nb)
                def _():
                    start_gather(i + 1, q)

                wait_gather(p)
                compute(p)
                start_write(i, p)
            return carry

        lax.fori_loop(0, nb // 2, body, 0)
        wait_write((nb - 1) % 2)

    return _emb_ln(X, E_table, P_table, gamma, beta)


# speedup vs baseline: 1.6051x; 1.0038x over previous
import functools

import jax
import jax.numpy as jnp
from jax import lax
from jax.experimental import pallas as pl
from jax.experimental.pallas import tpu as pltpu
from jax.experimental.pallas import tpu_sc as plsc

_LANES = 16
_NC = 2
_NS = 16
_UNROLL = 1


def _allsum16(v):
    dnums = lax.GatherDimensionNumbers(
        offset_dims=(), collapsed_slice_dims=(0,), start_index_map=(0,))
    for k in (8, 4, 2, 1):
        idx = lax.iota(jnp.int32, _LANES) ^ k
        v = v + lax.gather(v, idx[:, None], dnums, slice_sizes=(1,),
                           mode=lax.GatherScatterMode.PROMISE_IN_BOUNDS)
    return v


def _rsqrt16(x):
    i = lax.bitcast_convert_type(x, jnp.int32)
    i = jnp.int32(0x5F3759DF) - lax.shift_right_logical(i, 1)
    y = lax.bitcast_convert_type(i, jnp.float32)
    for _ in range(1):
        y = y * (1.5 - 0.5 * x * y * y)
    return y


def kernel(X, E_table, P_table, gamma, beta):
    B, L = X.shape
    V, D = E_table.shape
    nd = D // _LANES
    NW = _NC * _NS
    nb = B // NW
    X = X.astype(jnp.int32)

    chunks = []
    off = 0
    while off < L:
        c = min(128, L - off)
        chunks.append((off, c))
        off += c

    mesh = plsc.VectorSubcoreMesh(
        core_axis_name="c", subcore_axis_name="s",
        num_cores=_NC, num_subcores=_NS)

    @functools.partial(
        pl.kernel,
        mesh=mesh,
        out_type=jax.ShapeDtypeStruct((B, L, D), jnp.float32),
        scratch_types=[
            pltpu.VMEM((nb, L), jnp.int32),
            pltpu.VMEM((L, D), jnp.float32),
            pltpu.VMEM((L, D), jnp.float32),
            pltpu.VMEM((L, D), jnp.float32),
            pltpu.VMEM((D,), jnp.float32),
            pltpu.VMEM((D,), jnp.float32),
            pltpu.SemaphoreType.DMA,
            pltpu.SemaphoreType.DMA,
            pltpu.SemaphoreType.DMA,
            pltpu.SemaphoreType.DMA,
        ],
    )
    def _emb_ln(x_hbm, e_hbm, p_hbm, g_hbm, b_hbm, out_hbm,
                idx_v, rows0, rows1, p_v, g_v, b_v,
                gsem0, gsem1, wsem0, wsem1):
        wid = lax.axis_index("s") * _NC + lax.axis_index("c")
        base = wid * nb

        pltpu.sync_copy(x_hbm.at[pl.ds(base, nb)], idx_v)
        pltpu.sync_copy(p_hbm.at[pl.ds(0, L)], p_v)
        pltpu.sync_copy(g_hbm, g_v)
        pltpu.sync_copy(b_hbm, b_v)
        gv = [g_v[pl.ds(j * _LANES, _LANES)] for j in range(nd)]
        bv = [b_v[pl.ds(j * _LANES, _LANES)] for j in range(nd)]

        rows = (rows0, rows1)
        gsem = (gsem0, gsem1)
        wsem = (wsem0, wsem1)

        def start_gather(i, p):
            for off, c in chunks:
                pltpu.async_copy(e_hbm.at[idx_v.at[i, pl.ds(off, c)]],
                                 rows[p].at[pl.ds(off, c)], gsem[p])

        def wait_gather(p):
            pltpu.make_async_copy(e_hbm.at[pl.ds(0, L)], rows[p],
                                  gsem[p]).wait()

        def start_write(i, p):
            pltpu.async_copy(rows[p], out_hbm.at[base + i], wsem[p])

        def wait_write(p):
            pltpu.make_async_copy(rows[p], out_hbm.at[0], wsem[p]).wait()

        def compute(p):
            rbuf = rows[p]

            @plsc.parallel_loop(0, L, unroll=_UNROLL)
            def row_body(r):
                xs = []
                s = None
                ss = None
                for j in range(nd):
                    e = rbuf[r, pl.ds(j * _LANES, _LANES)]
                    pj = p_v[r, pl.ds(j * _LANES, _LANES)]
                    x = e + pj
                    xs.append(x)
                    s = x if s is None else s + x
                    ss = x * x if ss is None else ss + x * x
                mean = _allsum16(s) * (1.0 / D)
                var = _allsum16(ss) * (1.0 / D) - mean * mean
                rstd = _rsqrt16(var + 1e-12)
                for j in range(nd):
                    o = (xs[j] - mean) * rstd * gv[j] + bv[j]
                    rbuf[r, pl.ds(j * _LANES, _LANES)] = o

        start_gather(0, 0)

        def body(i2, carry):
            for b2 in range(2):
                i = i2 * 2 + b2
                p = b2
                q = 1 - b2

                @pl.when(i >= 1)
                def _():
                    wait_write(q)

                @pl.when(i + 1 < nb)
                def _():
                    start_gather(i + 1, q)

                wait_gather(p)
                compute(p)
                start_write(i, p)
            return carry

        lax.fori_loop(0, nb // 2, body, 0)
        wait_write((nb - 1) % 2)

    return _emb_ln(X, E_table, P_table, gamma, beta)
